# baseline (speedup 1.0000x reference)
import functools

import jax
import jax.numpy as jnp
from jax import lax
from jax.experimental import pallas as pl
from jax.experimental.pallas import tpu as pltpu
from jax.experimental.pallas import tpu_sc as plsc

NN = 50000
EE = 800000
GG = 256
HH = 256
NSL = 8
WSL = 32
N8 = 50048
TPR = 3128
NCH = 16
EPT = EE // NCH
BK = 128
NB = 400
GB = 16
NG = NB // GB
BN = 2000
NBLK = NN // BN


NBUF = 4


def _sc_agg_body(z_hbm, src_hbm, dst_hbm, zeros_hbm, agg_hbm,
                 src_v, dst_v, gbufs, spmem, gsems, ssems, isems):
    c = lax.axis_index("c")
    s = lax.axis_index("s")
    row0 = s * TPR

    for r in range(4):
        q = c * 4 + r
        sbase = (s * 8 + c * 4 + r) * NG
        pltpu.sync_copy(zeros_hbm, spmem.at[pl.ds(row0, TPR)])
        plsc.subcore_barrier()
        pltpu.async_copy(src_hbm.at[sbase], src_v.at[0], isems.at[0])
        pltpu.async_copy(dst_hbm.at[s * NG], dst_v.at[0], isems.at[2])

        pltpu.sync_copy(src_hbm.at[sbase], src_v.at[0])
        pltpu.sync_copy(dst_hbm.at[s * NG], dst_v.at[0])
        for k in range(NBUF):
            pltpu.async_copy(z_hbm.at[src_v.at[0, k]], gbufs.at[k],
                             gsems.at[k])

        def step(j, carry):
            for k in range(NBUF):
                b = NBUF * j + k
                g = lax.div(b, GB)
                sl = lax.rem(g, 2)
                rb = lax.rem(b, GB)

                @pl.when(jnp.logical_and(rb == 0, g + 1 < NG))
                def _():
                    sl2 = lax.rem(g + 1, 2)
                    pltpu.async_copy(src_hbm.at[sbase + g + 1],
                                     src_v.at[sl2], isems.at[sl2])
                    pltpu.async_copy(dst_hbm.at[s * NG + g + 1],
                                     dst_v.at[sl2], isems.at[sl2 + 2])

                pltpu.make_async_copy(z_hbm.at[src_v.at[sl, rb]],
                                      gbufs.at[k], gsems.at[k]).wait()
                pltpu.async_copy(gbufs.at[k], spmem.at[dst_v.at[sl, rb]],
                                 ssems.at[k], add=True)

                @pl.when(b < NB - NBUF)
                def _():
                    bn = b + NBUF
                    gn = lax.div(bn, GB)
                    sln = lax.rem(gn, 2)
                    rbn = lax.rem(bn, GB)

                    @pl.when(rbn == 0)
                    def _():
                        pltpu.make_async_copy(src_hbm.at[sbase + gn],
                                              src_v.at[sln],
                                              isems.at[sln]).wait()
                        pltpu.make_async_copy(dst_hbm.at[s * NG + gn],
                                              dst_v.at[sln],
                                              isems.at[sln + 2]).wait()

                    pltpu.make_async_copy(gbufs.at[k],
                                          spmem.at[dst_v.at[sl, rb]],
                                          ssems.at[k]).wait()
                    pltpu.async_copy(z_hbm.at[src_v.at[sln, rbn]],
                                     gbufs.at[k], gsems.at[k])
            return carry

        lax.fori_loop(0, NB // NBUF, step, 0, unroll=False)
        for k in range(NBUF):
            b = NB - NBUF + k
            pltpu.make_async_copy(
                gbufs.at[k],
                spmem.at[dst_v.at[lax.rem(NG - 1, 2), lax.rem(b, GB)]],
                ssems.at[k]).wait()
        plsc.subcore_barrier()
        pltpu.sync_copy(spmem.at[pl.ds(row0, TPR)],
                        agg_hbm.at[pl.ds(row0, TPR), q])


def _sc_agg(z_flat, src_adj, dst_t, zeros):
    mesh = plsc.VectorSubcoreMesh(core_axis_name="c", subcore_axis_name="s")
    f = pl.kernel(
        _sc_agg_body,
        out_type=jax.ShapeDtypeStruct((N8, NSL, WSL), jnp.float32),
        mesh=mesh,
        scratch_types=[
            pltpu.VMEM((2, GB, BK), jnp.int32),
            pltpu.VMEM((2, GB, BK), jnp.int32),
            pltpu.VMEM((NBUF, BK, WSL), jnp.float32),
            pltpu.VMEM_SHARED((N8, WSL), jnp.float32),
            pltpu.SemaphoreType.DMA((NBUF,)),
            pltpu.SemaphoreType.DMA((NBUF,)),
            pltpu.SemaphoreType.DMA((4,)),
        ],
        compiler_params=pltpu.CompilerParams(use_tc_tiling_on_sc=False),
    )
    return f(z_flat, src_adj, dst_t, zeros)


_DOT = functools.partial(jax.lax.dot_general,
                         precision=jax.lax.Precision.HIGHEST,
                         preferred_element_type=jnp.float32)


def _mm(a, b):
    return _DOT(a, b, (((1,), (0,)), ((), ())))


def _pre_body(x_ref, w_ref, o_ref):
    o_ref[...] = _mm(x_ref[...], w_ref[...])


def _pre(x8, w1p):
    return pl.pallas_call(
        _pre_body,
        grid=(NBLK,),
        in_specs=[
            pl.BlockSpec((BN, 8), lambda i: (i, 0)),
            pl.BlockSpec((8, HH), lambda i: (0, 0)),
        ],
        out_specs=pl.BlockSpec((BN, HH), lambda i: (i, 0)),
        out_shape=jax.ShapeDtypeStruct((NN, HH), jnp.float32),
    )(x8, w1p)


def _mid_body(z_ref, a_ref, b1_ref, w2_ref, b2_ref, w1n_ref, o_ref):
    u = jnp.maximum(z_ref[...] + a_ref[...] + b1_ref[...], 0.0)
    h = jnp.maximum(_mm(u, w2_ref[...]) + b2_ref[...], 0.0)
    o_ref[...] = _mm(h, w1n_ref[...])


def _mid(z, agg, b1, w2, b2, w1n):
    return pl.pallas_call(
        _mid_body,
        grid=(NBLK,),
        in_specs=[
            pl.BlockSpec((BN, HH), lambda i: (i, 0)),
            pl.BlockSpec((BN, HH), lambda i: (i, 0)),
            pl.BlockSpec((1, HH), lambda i: (0, 0)),
            pl.BlockSpec((HH, HH), lambda i: (0, 0)),
            pl.BlockSpec((1, HH), lambda i: (0, 0)),
            pl.BlockSpec((HH, HH), lambda i: (0, 0)),
        ],
        out_specs=pl.BlockSpec((BN, HH), lambda i: (i, 0)),
        out_shape=jax.ShapeDtypeStruct((NN, HH), jnp.float32),
    )(z, agg, b1, w2, b2, w1n)


def _last_body(z_ref, a_ref, b_ref, b1_ref, w2_ref, b2_ref,
               l1w_ref, l1b_ref, l2w_ref, l2b_ref,
               sums_ref, cnts_ref, o_ref):
    i = pl.program_id(0)

    @pl.when(i == 0)
    def _():
        sums_ref[...] = jnp.zeros_like(sums_ref)
        cnts_ref[...] = jnp.zeros_like(cnts_ref)

    u = jnp.maximum(z_ref[...] + a_ref[...] + b1_ref[...], 0.0)
    h = jnp.maximum(_mm(u, w2_ref[...]) + b2_ref[...], 0.0)
    gid = lax.broadcasted_iota(jnp.int32, (1, GG), 1)
    oh = (b_ref[...] == gid).astype(jnp.float32)
    sums_ref[...] += _DOT(oh, h, (((0,), (0,)), ((), ())))
    cnts_ref[...] += jnp.sum(oh, axis=0, keepdims=True)

    @pl.when(i == NBLK - 1)
    def _():
        cnt = jnp.maximum(cnts_ref[...].reshape(GG, 1), 1.0)
        pooled = sums_ref[...] / cnt
        a = jnp.maximum(_mm(pooled, l1w_ref[...]) + l1b_ref[...], 0.0)
        o_ref[...] = _mm(a, l2w_ref[...]) + l2b_ref[...]


def _last_pool_head(z, agg, batch2d, b1, w2, b2, l1_w, l1_b, l2_wp, l2_bp):
    outs = pl.pallas_call(
        _last_body,
        grid=(NBLK,),
        in_specs=[
            pl.BlockSpec((BN, HH), lambda i: (i, 0)),
            pl.BlockSpec((BN, HH), lambda i: (i, 0)),
            pl.BlockSpec((BN, 1), lambda i: (i, 0)),
            pl.BlockSpec((1, HH), lambda i: (0, 0)),
            pl.BlockSpec((HH, HH), lambda i: (0, 0)),
            pl.BlockSpec((1, HH), lambda i: (0, 0)),
            pl.BlockSpec((HH, 128), lambda i: (0, 0)),
            pl.BlockSpec((1, 128), lambda i: (0, 0)),
            pl.BlockSpec((128, 128), lambda i: (0, 0)),
            pl.BlockSpec((1, 128), lambda i: (0, 0)),
        ],
        out_specs=[
            pl.BlockSpec((GG, HH), lambda i: (0, 0)),
            pl.BlockSpec((1, GG), lambda i: (0, 0)),
            pl.BlockSpec((GG, 128), lambda i: (0, 0)),
        ],
        out_shape=[
            jax.ShapeDtypeStruct((GG, HH), jnp.float32),
            jax.ShapeDtypeStruct((1, GG), jnp.float32),
            jax.ShapeDtypeStruct((GG, 128), jnp.float32),
        ],
    )(z, agg, batch2d, b1, w2, b2, l1_w, l1_b, l2_wp, l2_bp)
    return outs[2]


def kernel(x, edge_index, batch,
           c1_w1, c1_b1, c1_w2, c1_b2,
           c2_w1, c2_b1, c2_w2, c2_b2,
           c3_w1, c3_b1, c3_w2, c3_b2,
           c4_w1, c4_b1, c4_w2, c4_b2,
           l1_w, l1_b, l2_w, l2_b):
    f32 = jnp.float32
    x8 = jnp.pad(x, ((0, 0), (0, 1)))
    w1p = jnp.pad(c1_w1, ((0, 1), (0, 0)))

    src = edge_index[0].reshape(NCH, EPT)
    dst = edge_index[1].reshape(NCH, EPT)
    src_p = jnp.pad(src, ((0, 0), (0, NB * BK - EPT)))
    dst_p = jnp.pad(dst, ((0, 0), (0, NB * BK - EPT)),
                    constant_values=NN)
    qoff = jnp.arange(8, dtype=jnp.int32).reshape(1, 2, 4, 1)
    src_adj = (src_p[:, None, None, :] * 8 + qoff).reshape(NCH * 8 * NG,
                                                           GB, BK)
    dst_t = dst_p.reshape(NCH * NG, GB, BK)
    zeros = jnp.zeros((TPR, WSL), dtype=f32)

    batch2d = batch.reshape(NN, 1)
    b1s = [c1_b1.reshape(1, HH), c2_b1.reshape(1, HH),
           c3_b1.reshape(1, HH), c4_b1.reshape(1, HH)]
    b2s = [c1_b2.reshape(1, HH), c2_b2.reshape(1, HH),
           c3_b2.reshape(1, HH), c4_b2.reshape(1, HH)]
    w2s = [c1_w2, c2_w2, c3_w2, c4_w2]
    w1n = [c2_w1, c3_w1, c4_w1]
    l2_wp = jnp.pad(l2_w, ((0, 0), (0, 128 - l2_w.shape[1])))
    l2_bp = jnp.pad(l2_b, ((0, 128 - l2_b.shape[0]),)).reshape(1, 128)

    z = _pre(x8, w1p)
    for l in range(4):
        agg = _sc_agg(z.reshape(NSL * NN, WSL), src_adj, dst_t, zeros)
        agg = agg.reshape(N8, HH)
        if l < 3:
            z = _mid(z, agg, b1s[l], w2s[l], b2s[l], w1n[l])
        else:
            out = _last_pool_head(z, agg, batch2d, b1s[l], w2s[l], b2s[l],
                                  l1_w, l1_b.reshape(1, 128), l2_wp, l2_bp)
    return out[:, :3]


# speedup vs baseline: 1.7257x; 1.7257x over previous
import functools

import jax
import jax.numpy as jnp
from jax import lax
from jax.experimental import pallas as pl
from jax.experimental.pallas import tpu as pltpu
from jax.experimental.pallas import tpu_sc as plsc

NN = 50000
EE = 800000
GG = 256
HH = 256
NSL = 8
WSL = 32
N8 = 50048
TPR = 3128
NCH = 16
EPT = EE // NCH
BK = 128
NB = 392
GB = 28
NG = NB // GB
BN = 2000
NBLK = NN // BN


NBUF = 4


def _sc_agg_body(z_hbm, src_hbm, dst_hbm, zeros_hbm, agg_hbm,
                 src_v, dst_v, gbufs, spmem, gsems, ssems, isems):
    c = lax.axis_index("c")
    s = lax.axis_index("s")
    row0 = s * TPR

    for r in range(4):
        q = c * 4 + r
        sbase = (s * 8 + c * 4 + r) * NG
        pltpu.sync_copy(zeros_hbm, spmem.at[pl.ds(row0, TPR)])
        plsc.subcore_barrier()
        pltpu.async_copy(src_hbm.at[sbase], src_v.at[0], isems.at[0])
        pltpu.async_copy(dst_hbm.at[s * NG], dst_v.at[0], isems.at[2])

        pltpu.sync_copy(src_hbm.at[sbase], src_v.at[0])
        pltpu.sync_copy(dst_hbm.at[s * NG], dst_v.at[0])
        for k in range(NBUF):
            pltpu.async_copy(z_hbm.at[src_v.at[0, k]], gbufs.at[k],
                             gsems.at[k])

        def step(j, carry):
            for k in range(NBUF):
                b = NBUF * j + k
                g = lax.div(b, GB)
                sl = lax.rem(g, 2)
                rb = lax.rem(b, GB)

                @pl.when(jnp.logical_and(rb == 0, g + 1 < NG))
                def _():
                    sl2 = lax.rem(g + 1, 2)
                    pltpu.async_copy(src_hbm.at[sbase + g + 1],
                                     src_v.at[sl2], isems.at[sl2])
                    pltpu.async_copy(dst_hbm.at[s * NG + g + 1],
                                     dst_v.at[sl2], isems.at[sl2 + 2])

                pltpu.make_async_copy(z_hbm.at[src_v.at[sl, rb]],
                                      gbufs.at[k], gsems.at[k]).wait()
                pltpu.async_copy(gbufs.at[k], spmem.at[dst_v.at[sl, rb]],
                                 ssems.at[k], add=True)

                @pl.when(b < NB - NBUF)
                def _():
                    bn = b + NBUF
                    gn = lax.div(bn, GB)
                    sln = lax.rem(gn, 2)
                    rbn = lax.rem(bn, GB)

                    @pl.when(rbn == 0)
                    def _():
                        pltpu.make_async_copy(src_hbm.at[sbase + gn],
                                              src_v.at[sln],
                                              isems.at[sln]).wait()
                        pltpu.make_async_copy(dst_hbm.at[s * NG + gn],
                                              dst_v.at[sln],
                                              isems.at[sln + 2]).wait()

                    pltpu.make_async_copy(gbufs.at[k],
                                          spmem.at[dst_v.at[sl, rb]],
                                          ssems.at[k]).wait()
                    pltpu.async_copy(z_hbm.at[src_v.at[sln, rbn]],
                                     gbufs.at[k], gsems.at[k])
            return carry

        lax.fori_loop(0, NB // NBUF, step, 0, unroll=False)
        for k in range(NBUF):
            b = NB - NBUF + k
            pltpu.make_async_copy(
                gbufs.at[k],
                spmem.at[dst_v.at[lax.rem(NG - 1, 2), lax.rem(b, GB)]],
                ssems.at[k]).wait()
        plsc.subcore_barrier()
        pltpu.sync_copy(spmem.at[pl.ds(row0, TPR)],
                        agg_hbm.at[pl.ds(row0, TPR), q])


def _sc_agg(z_flat, src_adj, dst_t, zeros):
    mesh = plsc.VectorSubcoreMesh(core_axis_name="c", subcore_axis_name="s")
    f = pl.kernel(
        _sc_agg_body,
        out_type=jax.ShapeDtypeStruct((N8, NSL, WSL), jnp.float32),
        mesh=mesh,
        scratch_types=[
            pltpu.VMEM((2, GB, BK), jnp.int32),
            pltpu.VMEM((2, GB, BK), jnp.int32),
            pltpu.VMEM((NBUF, BK, WSL), jnp.float32),
            pltpu.VMEM_SHARED((N8, WSL), jnp.float32),
            pltpu.SemaphoreType.DMA((NBUF,)),
            pltpu.SemaphoreType.DMA((NBUF,)),
            pltpu.SemaphoreType.DMA((4,)),
        ],
        compiler_params=pltpu.CompilerParams(use_tc_tiling_on_sc=False),
    )
    return f(z_flat, src_adj, dst_t, zeros)


_DOT = functools.partial(jax.lax.dot_general,
                         precision=jax.lax.Precision.DEFAULT,
                         preferred_element_type=jnp.float32)


def _mm(a, b):
    return _DOT(a, b, (((1,), (0,)), ((), ())))


def _pre_body(x_ref, w_ref, o_ref):
    o_ref[...] = _mm(x_ref[...], w_ref[...])


def _pre(x8, w1p):
    return pl.pallas_call(
        _pre_body,
        grid=(NBLK,),
        in_specs=[
            pl.BlockSpec((BN, 8), lambda i: (i, 0)),
            pl.BlockSpec((8, HH), lambda i: (0, 0)),
        ],
        out_specs=pl.BlockSpec((BN, HH), lambda i: (i, 0)),
        out_shape=jax.ShapeDtypeStruct((NN, HH), jnp.float32),
    )(x8, w1p)


def _mid_body(z_ref, a_ref, b1_ref, w2_ref, b2_ref, w1n_ref, o_ref):
    u = jnp.maximum(z_ref[...] + a_ref[...] + b1_ref[...], 0.0)
    h = jnp.maximum(_mm(u, w2_ref[...]) + b2_ref[...], 0.0)
    o_ref[...] = _mm(h, w1n_ref[...])


def _mid(z, agg, b1, w2, b2, w1n):
    return pl.pallas_call(
        _mid_body,
        grid=(NBLK,),
        in_specs=[
            pl.BlockSpec((BN, HH), lambda i: (i, 0)),
            pl.BlockSpec((BN, HH), lambda i: (i, 0)),
            pl.BlockSpec((1, HH), lambda i: (0, 0)),
            pl.BlockSpec((HH, HH), lambda i: (0, 0)),
            pl.BlockSpec((1, HH), lambda i: (0, 0)),
            pl.BlockSpec((HH, HH), lambda i: (0, 0)),
        ],
        out_specs=pl.BlockSpec((BN, HH), lambda i: (i, 0)),
        out_shape=jax.ShapeDtypeStruct((NN, HH), jnp.float32),
    )(z, agg, b1, w2, b2, w1n)


def _last_body(z_ref, a_ref, b_ref, b1_ref, w2_ref, b2_ref,
               l1w_ref, l1b_ref, l2w_ref, l2b_ref,
               sums_ref, cnts_ref, o_ref):
    i = pl.program_id(0)

    @pl.when(i == 0)
    def _():
        sums_ref[...] = jnp.zeros_like(sums_ref)
        cnts_ref[...] = jnp.zeros_like(cnts_ref)

    u = jnp.maximum(z_ref[...] + a_ref[...] + b1_ref[...], 0.0)
    h = jnp.maximum(_mm(u, w2_ref[...]) + b2_ref[...], 0.0)
    gid = lax.broadcasted_iota(jnp.int32, (1, GG), 1)
    oh = (b_ref[...] == gid).astype(jnp.float32)
    sums_ref[...] += _DOT(oh, h, (((0,), (0,)), ((), ())))
    cnts_ref[...] += jnp.sum(oh, axis=0, keepdims=True)

    @pl.when(i == NBLK - 1)
    def _():
        cnt = jnp.maximum(cnts_ref[...].reshape(GG, 1), 1.0)
        pooled = sums_ref[...] / cnt
        a = jnp.maximum(_mm(pooled, l1w_ref[...]) + l1b_ref[...], 0.0)
        o_ref[...] = _mm(a, l2w_ref[...]) + l2b_ref[...]


def _last_pool_head(z, agg, batch2d, b1, w2, b2, l1_w, l1_b, l2_wp, l2_bp):
    outs = pl.pallas_call(
        _last_body,
        grid=(NBLK,),
        in_specs=[
            pl.BlockSpec((BN, HH), lambda i: (i, 0)),
            pl.BlockSpec((BN, HH), lambda i: (i, 0)),
            pl.BlockSpec((BN, 1), lambda i: (i, 0)),
            pl.BlockSpec((1, HH), lambda i: (0, 0)),
            pl.BlockSpec((HH, HH), lambda i: (0, 0)),
            pl.BlockSpec((1, HH), lambda i: (0, 0)),
            pl.BlockSpec((HH, 128), lambda i: (0, 0)),
            pl.BlockSpec((1, 128), lambda i: (0, 0)),
            pl.BlockSpec((128, 128), lambda i: (0, 0)),
            pl.BlockSpec((1, 128), lambda i: (0, 0)),
        ],
        out_specs=[
            pl.BlockSpec((GG, HH), lambda i: (0, 0)),
            pl.BlockSpec((1, GG), lambda i: (0, 0)),
            pl.BlockSpec((GG, 128), lambda i: (0, 0)),
        ],
        out_shape=[
            jax.ShapeDtypeStruct((GG, HH), jnp.float32),
            jax.ShapeDtypeStruct((1, GG), jnp.float32),
            jax.ShapeDtypeStruct((GG, 128), jnp.float32),
        ],
    )(z, agg, batch2d, b1, w2, b2, l1_w, l1_b, l2_wp, l2_bp)
    return outs[2]


def kernel(x, edge_index, batch,
           c1_w1, c1_b1, c1_w2, c1_b2,
           c2_w1, c2_b1, c2_w2, c2_b2,
           c3_w1, c3_b1, c3_w2, c3_b2,
           c4_w1, c4_b1, c4_w2, c4_b2,
           l1_w, l1_b, l2_w, l2_b):
    f32 = jnp.float32
    x8 = jnp.pad(x, ((0, 0), (0, 1)))
    w1p = jnp.pad(c1_w1, ((0, 1), (0, 0)))

    src = edge_index[0].reshape(NCH, EPT)
    dst = edge_index[1].reshape(NCH, EPT)
    src_p = jnp.pad(src, ((0, 0), (0, NB * BK - EPT)))
    dst_p = jnp.pad(dst, ((0, 0), (0, NB * BK - EPT)),
                    constant_values=NN)
    qoff = jnp.arange(8, dtype=jnp.int32).reshape(1, 2, 4, 1)
    src_adj = (src_p[:, None, None, :] * 8 + qoff).reshape(NCH * 8 * NG,
                                                           GB, BK)
    dst_t = dst_p.reshape(NCH * NG, GB, BK)
    zeros = jnp.zeros((TPR, WSL), dtype=f32)

    batch2d = batch.reshape(NN, 1)
    b1s = [c1_b1.reshape(1, HH), c2_b1.reshape(1, HH),
           c3_b1.reshape(1, HH), c4_b1.reshape(1, HH)]
    b2s = [c1_b2.reshape(1, HH), c2_b2.reshape(1, HH),
           c3_b2.reshape(1, HH), c4_b2.reshape(1, HH)]
    w2s = [c1_w2, c2_w2, c3_w2, c4_w2]
    w1n = [c2_w1, c3_w1, c4_w1]
    l2_wp = jnp.pad(l2_w, ((0, 0), (0, 128 - l2_w.shape[1])))
    l2_bp = jnp.pad(l2_b, ((0, 128 - l2_b.shape[0]),)).reshape(1, 128)

    z = _pre(x8, w1p)
    for l in range(4):
        agg = _sc_agg(z.reshape(NSL * NN, WSL), src_adj, dst_t, zeros)
        agg = agg.reshape(N8, HH)
        if l < 3:
            z = _mid(z, agg, b1s[l], w2s[l], b2s[l], w1n[l])
        else:
            out = _last_pool_head(z, agg, batch2d, b1s[l], w2s[l], b2s[l],
                                  l1_w, l1_b.reshape(1, 128), l2_wp, l2_bp)
    return out[:, :3]
